# baseline (speedup 1.0000x reference)
import functools

import jax
import jax.numpy as jnp
from jax import lax
from jax.experimental import pallas as pl
from jax.experimental.pallas import tpu as pltpu
from jax.experimental.pallas import tpu_sc as plsc

SIGMA = 1.0
C_NUM = 512
Z_CHANNELS = 64
TILE = 576


def _vq_kernel(x_ref, c_ref, soft_ref, sym_ref, symd_ref, phi_ref,
               caug_ref, kidx_ref):
    x = x_ref[...]
    c = c_ref[...]

    @pl.when(pl.program_id(0) == 0)
    def _build_invariants():
        cn = jnp.sum(c * c, axis=1, keepdims=True)
        ones_k = jnp.ones((c.shape[0], 1), jnp.float32)
        caug_ref[...] = jnp.concatenate([-2.0 * c, cn, ones_k], axis=1)
        kidx_ref[...] = jax.lax.broadcasted_iota(
            jnp.int32, kidx_ref.shape, 1)

    c_aug = caug_ref[...]
    kidx = kidx_ref[...]

    xn = jnp.sum(x * x, axis=1, keepdims=True)
    ones_t = jnp.ones((x.shape[0], 1), jnp.float32)
    x_aug = jnp.concatenate([x, ones_t, xn], axis=1)
    d2 = jax.lax.dot_general(
        x_aug, c_aug, (((1,), (1,)), ((), ())),
        preferred_element_type=jnp.float32,
        precision=jax.lax.Precision.HIGHEST)
    d = jnp.sqrt(jnp.maximum(d2, 0.0))

    mind = jnp.min(d, axis=1, keepdims=True)
    e = jnp.exp(SIGMA * (mind - d))
    phis = e * (1.0 / jnp.sum(e, axis=1, keepdims=True))
    phi_ref[...] = phis

    kb = kidx[:1]
    sym = jnp.min(jnp.where(d == mind, kb, C_NUM), axis=1)
    sym_ref[0, 0] = sym.reshape(24, 24)
    symd_ref[0, 0] = sym

    soft = jax.lax.dot_general(
        phis, c, (((1,), (0,)), ((), ())),
        preferred_element_type=jnp.float32)
    soft_ref[...] = soft


@jax.jit
def kernel(data, centers):
    b, c, h, w = data.shape
    n = b * h * w
    k = centers.shape[0]
    nb = n // TILE
    x = jnp.transpose(data, (0, 2, 3, 1)).reshape(n, c)

    out_shapes = (
        jax.ShapeDtypeStruct((n, c), jnp.float32),
        jax.ShapeDtypeStruct((nb, 1, h, w), jnp.int32),
        jax.ShapeDtypeStruct((nb, 1, TILE), jnp.int32),
        jax.ShapeDtypeStruct((n, k), jnp.float32),
    )
    tok = lambda cols: pl.BlockSpec((TILE, cols), lambda i: (i, 0))
    soft, sym, symd, phis = pl.pallas_call(
        _vq_kernel,
        grid=(nb,),
        in_specs=[
            tok(c),
            pl.BlockSpec((k, c), lambda i: (0, 0)),
        ],
        out_specs=(
            tok(c),
            pl.BlockSpec((1, 1, h, w), lambda i: (i, 0, 0, 0)),
            pl.BlockSpec((1, 1, TILE), lambda i: (i, 0, 0)),
            tok(k),
        ),
        out_shape=out_shapes,
        scratch_shapes=[
            pltpu.VMEM((k, c + 2), jnp.float32),
            pltpu.VMEM((8, k), jnp.int32),
        ],
    )(x, centers)

    info = plsc.get_sparse_core_info()
    nw = info.num_cores * info.num_subcores
    b_per_w = n // nw
    mesh = plsc.VectorSubcoreMesh(core_axis_name="c", subcore_axis_name="s")

    @functools.partial(
        pl.kernel, mesh=mesh,
        out_type=[
            jax.ShapeDtypeStruct((n, 128), jnp.float32),
            jax.ShapeDtypeStruct((n, 128), jnp.float32),
        ],
        scratch_types=[
            pltpu.VMEM((b_per_w,), jnp.int32),
            pltpu.VMEM((b_per_w, 128), jnp.float32),
            pltpu.SemaphoreType.DMA,
        ],
    )
    def _sc_gather(table_hbm, idx_hbm, hard_hbm, zbar_hbm, idx_v, rows_v, sem):
        wid = lax.axis_index("s") * info.num_cores + lax.axis_index("c")
        base = wid * b_per_w
        pltpu.sync_copy(idx_hbm.at[pl.ds(base, b_per_w)], idx_v)
        pltpu.async_copy(table_hbm.at[idx_v], rows_v, sem).wait()
        pltpu.sync_copy(rows_v, hard_hbm.at[pl.ds(base, b_per_w)])
        pltpu.sync_copy(rows_v, zbar_hbm.at[pl.ds(base, b_per_w)])

    table128 = jnp.pad(centers, ((0, 0), (0, 128 - c)))
    hard128, zbar128 = _sc_gather(table128, symd.reshape(n))
    hard, zbar = hard128[:, :c], zbar128[:, :c]

    def to_bchw(a, ch):
        return jnp.transpose(a.reshape(b, h, w, ch), (0, 3, 1, 2))

    return (to_bchw(zbar, c), to_bchw(soft, c), to_bchw(hard, c),
            sym, to_bchw(phis, k))


# speedup vs baseline: 2.0797x; 2.0797x over previous
import jax
import jax.numpy as jnp
from jax.experimental import pallas as pl
from jax.experimental.pallas import tpu as pltpu

SIGMA = 1.0
C_NUM = 512
Z_CHANNELS = 64
TILE = 576


def _vq_kernel(x_ref, c_ref, zbar_ref, soft_ref, hard_ref, sym_ref, phi_ref,
               caug_ref, kidx_ref):
    x = x_ref[...]
    c = c_ref[...]

    @pl.when(pl.program_id(0) == 0)
    def _build_invariants():
        cn = jnp.sum(c * c, axis=1, keepdims=True)
        ones_k = jnp.ones((c.shape[0], 1), jnp.float32)
        caug_ref[...] = jnp.concatenate([-2.0 * c, cn, ones_k], axis=1)
        kidx_ref[...] = jax.lax.broadcasted_iota(
            jnp.int32, kidx_ref.shape, 1)

    c_aug = caug_ref[...]
    kidx = kidx_ref[...]

    xn = jnp.sum(x * x, axis=1, keepdims=True)
    ones_t = jnp.ones((x.shape[0], 1), jnp.float32)
    x_aug = jnp.concatenate([x, ones_t, xn], axis=1)
    d2 = jax.lax.dot_general(
        x_aug, c_aug, (((1,), (1,)), ((), ())),
        preferred_element_type=jnp.float32,
        precision=jax.lax.Precision.HIGHEST)
    d = jnp.sqrt(jnp.maximum(d2, 0.0))

    mind = jnp.min(d, axis=1, keepdims=True)
    e = jnp.exp(SIGMA * (mind - d))
    phis = e * (1.0 / jnp.sum(e, axis=1, keepdims=True))
    phi_ref[...] = phis

    kb = kidx[:1]
    sym = jnp.min(jnp.where(d == mind, kb, C_NUM), axis=1)
    sym_ref[0, 0] = sym.reshape(24, 24)

    soft = jax.lax.dot_general(
        phis, c, (((1,), (0,)), ((), ())),
        preferred_element_type=jnp.float32)
    soft_ref[...] = soft

    onehot = (kb == sym[:, None]).astype(jnp.float32)
    hard = jax.lax.dot_general(
        onehot, c, (((1,), (0,)), ((), ())),
        preferred_element_type=jnp.float32)
    hard_ref[...] = hard

    zbar_ref[...] = soft + (hard - soft)


@jax.jit
def kernel(data, centers):
    b, c, h, w = data.shape
    n = b * h * w
    k = centers.shape[0]
    nb = n // TILE
    x = jnp.transpose(data, (0, 2, 3, 1)).reshape(n, c)

    out_shapes = (
        jax.ShapeDtypeStruct((n, c), jnp.float32),
        jax.ShapeDtypeStruct((n, c), jnp.float32),
        jax.ShapeDtypeStruct((n, c), jnp.float32),
        jax.ShapeDtypeStruct((nb, 1, h, w), jnp.int32),
        jax.ShapeDtypeStruct((n, k), jnp.float32),
    )
    tok = lambda cols: pl.BlockSpec((TILE, cols), lambda i: (i, 0))
    zbar, soft, hard, sym, phis = pl.pallas_call(
        _vq_kernel,
        grid=(nb,),
        in_specs=[
            tok(c),
            pl.BlockSpec((k, c), lambda i: (0, 0)),
        ],
        out_specs=(
            tok(c), tok(c), tok(c),
            pl.BlockSpec((1, 1, h, w), lambda i: (i, 0, 0, 0)),
            tok(k),
        ),
        out_shape=out_shapes,
        scratch_shapes=[
            pltpu.VMEM((k, c + 2), jnp.float32),
            pltpu.VMEM((8, k), jnp.int32),
        ],
    )(x, centers)

    def to_bchw(a, ch):
        return jnp.transpose(a.reshape(b, h, w, ch), (0, 3, 1, 2))

    return (to_bchw(zbar, c), to_bchw(soft, c), to_bchw(hard, c),
            sym, to_bchw(phis, k))


# speedup vs baseline: 2.1751x; 1.0459x over previous
import jax
import jax.numpy as jnp
from jax.experimental import pallas as pl
from jax.experimental.pallas import tpu as pltpu

SIGMA = 1.0
C_NUM = 512
Z_CHANNELS = 64
TILE = 1152


def _vq_kernel(x_ref, c_ref, zbar_ref, soft_ref, hard_ref, sym_ref, phi_ref,
               caug_ref, kidx_ref):
    x = x_ref[...]
    c = c_ref[...]

    @pl.when(pl.program_id(0) == 0)
    def _build_invariants():
        cn = jnp.sum(c * c, axis=1, keepdims=True)
        ones_k = jnp.ones((c.shape[0], 1), jnp.float32)
        caug_ref[...] = jnp.concatenate([-2.0 * c, cn, ones_k], axis=1)
        kidx_ref[...] = jax.lax.broadcasted_iota(
            jnp.int32, kidx_ref.shape, 1)

    c_aug = caug_ref[...]
    kidx = kidx_ref[...]

    xn = jnp.sum(x * x, axis=1, keepdims=True)
    ones_t = jnp.ones((x.shape[0], 1), jnp.float32)
    x_aug = jnp.concatenate([x, ones_t, xn], axis=1)
    d2 = jax.lax.dot_general(
        x_aug, c_aug, (((1,), (1,)), ((), ())),
        preferred_element_type=jnp.float32,
        precision=jax.lax.Precision.HIGHEST)
    d = jnp.sqrt(jnp.maximum(d2, 0.0))

    mind = jnp.min(d, axis=1, keepdims=True)
    e = jnp.exp(SIGMA * (mind - d))
    phis = e * (1.0 / jnp.sum(e, axis=1, keepdims=True))
    phi_ref[...] = phis

    kb = kidx[:1]
    sym = jnp.min(jnp.where(d == mind, kb, C_NUM), axis=1)
    sym_ref[0] = sym.reshape(2, 24, 24)

    soft = jax.lax.dot_general(
        phis, c, (((1,), (0,)), ((), ())),
        preferred_element_type=jnp.float32)
    soft_ref[...] = soft

    onehot = (kb == sym[:, None]).astype(jnp.float32)
    hard = jax.lax.dot_general(
        onehot, c, (((1,), (0,)), ((), ())),
        preferred_element_type=jnp.float32)
    hard_ref[...] = hard

    zbar_ref[...] = soft + (hard - soft)


@jax.jit
def kernel(data, centers):
    b, c, h, w = data.shape
    n = b * h * w
    k = centers.shape[0]
    nb = n // TILE
    x = jnp.transpose(data, (0, 2, 3, 1)).reshape(n, c)

    out_shapes = (
        jax.ShapeDtypeStruct((n, c), jnp.float32),
        jax.ShapeDtypeStruct((n, c), jnp.float32),
        jax.ShapeDtypeStruct((n, c), jnp.float32),
        jax.ShapeDtypeStruct((nb, 2, h, w), jnp.int32),
        jax.ShapeDtypeStruct((n, k), jnp.float32),
    )
    tok = lambda cols: pl.BlockSpec((TILE, cols), lambda i: (i, 0))
    zbar, soft, hard, sym, phis = pl.pallas_call(
        _vq_kernel,
        grid=(nb,),
        in_specs=[
            tok(c),
            pl.BlockSpec((k, c), lambda i: (0, 0)),
        ],
        out_specs=(
            tok(c), tok(c), tok(c),
            pl.BlockSpec((1, 2, h, w), lambda i: (i, 0, 0, 0)),
            tok(k),
        ),
        out_shape=out_shapes,
        scratch_shapes=[
            pltpu.VMEM((k, c + 2), jnp.float32),
            pltpu.VMEM((8, k), jnp.int32),
        ],
    )(x, centers)

    def to_bchw(a, ch):
        return jnp.transpose(a.reshape(b, h, w, ch), (0, 3, 1, 2))

    return (to_bchw(zbar, c), to_bchw(soft, c), to_bchw(hard, c),
            sym.reshape(b, 1, h, w), to_bchw(phis, k))
